# baseline (speedup 1.0000x reference)
import functools

import jax
import jax.numpy as jnp
import numpy as np
from jax import lax
from jax.experimental import pallas as pl
from jax.experimental.pallas import tpu as pltpu
from jax.experimental.pallas import tpu_sc as plsc

N_TOKENS = 4608
N_HALF = N_TOKENS // 2
BLK = 576
N_BLOCKS = N_HALF // BLK
K_CAND = 2
N_CODES = 1024
D_LAT = 64

_f32 = jnp.float32
_bf16 = jnp.bfloat16


def _mm1(a, b):
    return lax.dot_general(a.astype(_bf16), b.astype(_bf16),
                           (((1,), (1,)), ((), ())),
                           preferred_element_type=_f32)


def _c(v):
    return jnp.float32(v)


def _erfc(w):
    one = _c(1.0)
    ax = jnp.abs(w)
    z = w * w
    p = z * _c(7.85386146e-05)
    for c in (-0.000801019371, 0.00518832775, -0.0268538129, 0.112835854,
              -0.37612626):
        p = (p + _c(c)) * z
    p = p + _c(1.12837911)
    small = one - w * p
    nz = -z
    e = jnp.exp(nz)
    q = e * (one / ax)
    r = one / z
    pa = r * _c(0.0232682)
    for c in (-0.138703942, 0.368742466, -0.582473278, 0.621000469,
              -0.494451523, 0.340488, -0.274112701):
        pa = (pa + _c(c)) * r
    pa = pa + _c(0.563825965)
    pb = r * _c(-10.477664)
    for c in (12.9772, -7.49551868, 2.92101908, -1.01526523, 0.42184633,
              -0.282076746):
        pb = (pb + _c(c)) * r
    pb = pb + _c(0.564189494)
    big = q * jnp.where(ax < _c(2.0), pa, pb)
    big = jnp.where(nz < _c(-88.7228394), _c(0.0), big)
    big = jnp.where(w < _c(0.0), _c(2.0) - big, big)
    return jnp.where(ax < one, small, big)


def _gelu(x):
    return (x * _c(0.5)) * _erfc((-x) * _c(0.70710676908493042))


def _encode_topk_body(x_ref, w1_ref, b1_ref, w2_ref, b2_ref, wl_ref, bl_ref,
                      cb_ref, ze_ref, cand_ref):
    x = x_ref[...]
    h = _gelu(_mm1(x, w1_ref[...]) + b1_ref[...])
    h = _gelu(_mm1(h, w2_ref[...]) + b2_ref[...])
    z_e = _mm1(h, wl_ref[...]) + bl_ref[...]
    ze_ref[...] = z_e

    cb = cb_ref[...]
    cb2 = cb * cb
    c2h = cb2.astype(_bf16)
    c2l = (cb2 - c2h.astype(_f32)).astype(_bf16)
    ones = jnp.ones((1, 2 * D_LAT), _bf16)
    cn2 = lax.dot_general(ones, jnp.concatenate([c2h, c2l], axis=1),
                          (((1,), (1,)), ((), ())),
                          preferred_element_type=_f32)
    zh = z_e.astype(_bf16)
    zl = (z_e - zh.astype(_f32)).astype(_bf16)
    ch = cb.astype(_bf16)
    cl = (cb - ch.astype(_f32)).astype(_bf16)
    za = jnp.concatenate([zh, zl, zh], axis=1)
    ca = jnp.concatenate([ch, ch, cl], axis=1)
    s = lax.dot_general(za, ca, (((1,), (1,)), ((), ())),
                        preferred_element_type=_f32)
    dist = cn2 - 2.0 * s
    iota = lax.broadcasted_iota(jnp.int32, (BLK, N_CODES), 1)
    big = jnp.int32(1 << 30)
    cands = []
    work = dist
    for j in range(K_CAND):
        m = jnp.min(work, axis=1, keepdims=True)
        idxj = jnp.min(jnp.where(work == m, iota, big), axis=1, keepdims=True)
        cands.append(idxj)
        if j + 1 < K_CAND:
            work = jnp.where(iota == idxj, jnp.float32(np.inf), work)
    cand_ref[...] = jnp.concatenate(cands, axis=1)


def _refine_decode_body(x_ref, ze_ref, rows_ref, cand_ref,
                        d1_ref, db1_ref, d2_ref, db2_ref, wo_ref, bo_ref,
                        zq_ref, rsum_ref, csum_ref):
    z_e = ze_ref[...]
    best_d = jnp.full((BLK, 1), np.inf, _f32)
    best_i = jnp.full((BLK, 1), 1 << 30, jnp.int32)
    z_q = jnp.zeros((BLK, D_LAT), _f32)
    for j in range(K_CAND):
        row = rows_ref[j]
        idx = cand_ref[:, j:j + 1]
        diff = z_e - row
        dj = jnp.sqrt(jnp.sum(diff * diff, axis=1, keepdims=True))
        better = (dj < best_d) | ((dj == best_d) & (idx < best_i))
        best_d = jnp.where(better, dj, best_d)
        best_i = jnp.where(better, idx, best_i)
        z_q = jnp.where(better, row, z_q)
    zq_ref[...] = z_q

    r = jax.nn.gelu(_mm1(z_q, d1_ref[...]) + db1_ref[...], approximate=True)
    r = jax.nn.gelu(_mm1(r, d2_ref[...]) + db2_ref[...], approximate=True)
    x_rec = _mm1(r, wo_ref[...]) + bo_ref[...]
    x = x_ref[...]
    dr = x_rec - x
    dc = z_q - z_e
    rpart = jnp.sum(dr * dr).reshape(1, 1)
    cpart = jnp.sum(dc * dc).reshape(1, 1)

    @pl.when(pl.program_id(0) == 0)
    def _init():
        rsum_ref[...] = rpart
        csum_ref[...] = cpart

    @pl.when(pl.program_id(0) != 0)
    def _acc():
        rsum_ref[...] += rpart
        csum_ref[...] += cpart


_sc_info = plsc.get_sparse_core_info()
_NC, _NS = _sc_info.num_cores, _sc_info.num_subcores
_NW = _NC * _NS
_B_GATHER = N_HALF * K_CAND
_BPW = _B_GATHER // _NW


_TBL_WORDS = N_CODES * D_LAT
_LANES = 16
_GROUPS = _BPW // _LANES


@functools.partial(
    pl.kernel,
    mesh=plsc.VectorSubcoreMesh(core_axis_name="c", subcore_axis_name="s"),
    compiler_params=pltpu.CompilerParams(needs_layout_passes=False),
    out_type=jax.ShapeDtypeStruct((_B_GATHER * D_LAT,), _f32),
    scratch_types=[
        pltpu.VMEM((_TBL_WORDS,), _f32),
        pltpu.VMEM((_BPW,), jnp.int32),
        pltpu.VMEM((_BPW * D_LAT,), _f32),
    ],
)
def _sc_gather(table_hbm, idx_hbm, out_hbm, table_v, idx_v, rows_v):
    wid = lax.axis_index("s") * _NC + lax.axis_index("c")
    pltpu.sync_copy(table_hbm, table_v)
    pltpu.sync_copy(idx_hbm.at[pl.ds(wid * _BPW, _BPW)], idx_v)
    def group(g, carry):
        iv = idx_v[pl.ds(g * _LANES, _LANES)]
        gb = g * (_LANES * D_LAT)
        for l in range(_LANES):
            tb = iv[l] * D_LAT
            ob = gb + l * D_LAT
            for q in range(0, D_LAT, _LANES):
                rows_v[pl.ds(ob + q, _LANES)] = table_v[pl.ds(tb + q, _LANES)]
        return carry

    lax.fori_loop(0, _GROUPS, group, 0)
    pltpu.sync_copy(rows_v, out_hbm.at[pl.ds(wid * (_BPW * D_LAT), _BPW * D_LAT)])


def kernel(x, enc_w1, enc_b1, enc_w2, enc_b2, lat_w, lat_b, codebook,
           dec_w1, dec_b1, dec_w2, dec_b2, out_w, out_b):
    B, S, F = x.shape
    xf = x.reshape(N_TOKENS, F)
    b1 = enc_b1.reshape(1, -1)
    b2 = enc_b2.reshape(1, -1)
    bl = lat_b.reshape(1, -1)
    db1 = dec_b1.reshape(1, -1)
    db2 = dec_b2.reshape(1, -1)
    bo = out_b.reshape(1, -1)

    full = lambda shp: pl.BlockSpec(shp, lambda t: tuple(0 for _ in shp))

    def encode(xh):
        return pl.pallas_call(
            _encode_topk_body,
            grid=(N_BLOCKS,),
            in_specs=[
                pl.BlockSpec((BLK, F), lambda t: (t, 0)),
                full((64, F)), full((1, 64)),
                full((128, 64)), full((1, 128)),
                full((D_LAT, 128)), full((1, D_LAT)),
                full((N_CODES, D_LAT)),
            ],
            out_specs=[
                pl.BlockSpec((BLK, D_LAT), lambda t: (t, 0)),
                pl.BlockSpec((BLK, K_CAND), lambda t: (t, 0)),
            ],
            out_shape=[
                jax.ShapeDtypeStruct((N_HALF, D_LAT), _f32),
                jax.ShapeDtypeStruct((N_HALF, K_CAND), jnp.int32),
            ],
        )(xh, enc_w1, b1, enc_w2, b2, lat_w, bl, codebook)

    def refine(xh, zeh, rowsh, candh):
        return pl.pallas_call(
            _refine_decode_body,
            grid=(N_BLOCKS,),
            in_specs=[
                pl.BlockSpec((BLK, F), lambda t: (t, 0)),
                pl.BlockSpec((BLK, D_LAT), lambda t: (t, 0)),
                pl.BlockSpec((K_CAND, BLK, D_LAT), lambda t: (0, t, 0)),
                pl.BlockSpec((BLK, K_CAND), lambda t: (t, 0)),
                full((64, D_LAT)), full((1, 64)),
                full((128, 64)), full((1, 128)),
                full((F, 128)), full((1, F)),
            ],
            out_specs=[
                pl.BlockSpec((BLK, D_LAT), lambda t: (t, 0)),
                pl.BlockSpec((1, 1), lambda t: (0, 0)),
                pl.BlockSpec((1, 1), lambda t: (0, 0)),
            ],
            out_shape=[
                jax.ShapeDtypeStruct((N_HALF, D_LAT), _f32),
                jax.ShapeDtypeStruct((1, 1), _f32),
                jax.ShapeDtypeStruct((1, 1), _f32),
            ],
        )(xh, zeh, rowsh, candh, dec_w1, db1, dec_w2, db2, out_w, bo)

    cb_flat = codebook.reshape(_TBL_WORDS)
    x1, x2 = xf[:N_HALF], xf[N_HALF:]
    ze1, cand1 = encode(x1)
    ze2, cand2 = encode(x2)
    rows1 = _sc_gather(cb_flat, cand1.T.reshape(_B_GATHER))
    rows2 = _sc_gather(cb_flat, cand2.T.reshape(_B_GATHER))
    rows1 = rows1.reshape(K_CAND, N_HALF, D_LAT)
    rows2 = rows2.reshape(K_CAND, N_HALF, D_LAT)
    zq1, rs1, cs1 = refine(x1, ze1, rows1, cand1)
    zq2, rs2, cs2 = refine(x2, ze2, rows2, cand2)
    zq = jnp.concatenate([zq1, zq2], axis=0)
    rsum = rs1 + rs2
    csum = cs1 + cs2

    z_latent = zq.reshape(B, S, D_LAT)
    recon_loss = rsum[0, 0] / jnp.float32(N_TOKENS * F)
    commit = csum[0, 0] / jnp.float32(N_TOKENS * D_LAT)
    loss = recon_loss + 0.25 * commit + 0.25 * commit
    return (z_latent, loss)


# speedup vs baseline: 1.1178x; 1.1178x over previous
import functools

import jax
import jax.numpy as jnp
import numpy as np
from jax import lax
from jax.experimental import pallas as pl
from jax.experimental.pallas import tpu as pltpu
from jax.experimental.pallas import tpu_sc as plsc

N_TOKENS = 4608
BLK = 576
N_BLOCKS = N_TOKENS // BLK
K_CAND = 2
N_CODES = 1024
D_LAT = 64

_f32 = jnp.float32
_bf16 = jnp.bfloat16


def _mm1(a, b):
    return lax.dot_general(a.astype(_bf16), b.astype(_bf16),
                           (((1,), (1,)), ((), ())),
                           preferred_element_type=_f32)


def _c(v):
    return jnp.float32(v)


def _erfc(w):
    one = _c(1.0)
    ax = jnp.abs(w)
    z = w * w
    p = z * _c(7.85386146e-05)
    for c in (-0.000801019371, 0.00518832775, -0.0268538129, 0.112835854,
              -0.37612626):
        p = (p + _c(c)) * z
    p = p + _c(1.12837911)
    small = one - w * p
    nz = -z
    e = jnp.exp(nz)
    q = e * (one / ax)
    r = one / z
    pa = r * _c(0.0232682)
    for c in (-0.138703942, 0.368742466, -0.582473278, 0.621000469,
              -0.494451523, 0.340488, -0.274112701):
        pa = (pa + _c(c)) * r
    pa = pa + _c(0.563825965)
    pb = r * _c(-10.477664)
    for c in (12.9772, -7.49551868, 2.92101908, -1.01526523, 0.42184633,
              -0.282076746):
        pb = (pb + _c(c)) * r
    pb = pb + _c(0.564189494)
    big = q * jnp.where(ax < _c(2.0), pa, pb)
    big = jnp.where(nz < _c(-88.7228394), _c(0.0), big)
    big = jnp.where(w < _c(0.0), _c(2.0) - big, big)
    return jnp.where(ax < one, small, big)


def _gelu(x):
    return (x * _c(0.5)) * _erfc((-x) * _c(0.70710676908493042))


def _encode_topk_body(x_ref, w1_ref, b1_ref, w2_ref, b2_ref, wl_ref, bl_ref,
                      cb_ref, ze_ref, cand_ref):
    x = x_ref[...]
    h = _gelu(_mm1(x, w1_ref[...]) + b1_ref[...])
    h = _gelu(_mm1(h, w2_ref[...]) + b2_ref[...])
    z_e = _mm1(h, wl_ref[...]) + bl_ref[...]
    ze_ref[...] = z_e

    cb = cb_ref[...]
    cb2 = cb * cb
    c2h = cb2.astype(_bf16)
    c2l = (cb2 - c2h.astype(_f32)).astype(_bf16)
    ones = jnp.ones((1, 2 * D_LAT), _bf16)
    cn2 = lax.dot_general(ones, jnp.concatenate([c2h, c2l], axis=1),
                          (((1,), (1,)), ((), ())),
                          preferred_element_type=_f32)
    zh = z_e.astype(_bf16)
    zl = (z_e - zh.astype(_f32)).astype(_bf16)
    ch = cb.astype(_bf16)
    cl = (cb - ch.astype(_f32)).astype(_bf16)
    za = jnp.concatenate([zh, zl, zh], axis=1)
    ca = jnp.concatenate([ch, ch, cl], axis=1)
    s = lax.dot_general(za, ca, (((1,), (1,)), ((), ())),
                        preferred_element_type=_f32)
    dist = cn2 - 2.0 * s
    iota = lax.broadcasted_iota(jnp.int32, (BLK, N_CODES), 1)
    big = jnp.int32(1 << 30)
    cands = []
    work = dist
    for j in range(K_CAND):
        m = jnp.min(work, axis=1, keepdims=True)
        idxj = jnp.min(jnp.where(work == m, iota, big), axis=1, keepdims=True)
        cands.append(idxj)
        if j + 1 < K_CAND:
            work = jnp.where(iota == idxj, jnp.float32(np.inf), work)
    cand_ref[...] = jnp.concatenate(cands, axis=1)


def _refine_decode_body(x_ref, ze_ref, rows_ref, cand_ref,
                        d1_ref, db1_ref, d2_ref, db2_ref, wo_ref, bo_ref,
                        zq_ref, rsum_ref, csum_ref):
    z_e = ze_ref[...]
    best_d = jnp.full((BLK, 1), np.inf, _f32)
    best_i = jnp.full((BLK, 1), 1 << 30, jnp.int32)
    z_q = jnp.zeros((BLK, D_LAT), _f32)
    for j in range(K_CAND):
        row = rows_ref[j]
        idx = cand_ref[:, j:j + 1]
        diff = z_e - row
        dj = jnp.sqrt(jnp.sum(diff * diff, axis=1, keepdims=True))
        better = (dj < best_d) | ((dj == best_d) & (idx < best_i))
        best_d = jnp.where(better, dj, best_d)
        best_i = jnp.where(better, idx, best_i)
        z_q = jnp.where(better, row, z_q)
    zq_ref[...] = z_q

    r = jax.nn.gelu(_mm1(z_q, d1_ref[...]) + db1_ref[...], approximate=True)
    r = jax.nn.gelu(_mm1(r, d2_ref[...]) + db2_ref[...], approximate=True)
    x_rec = _mm1(r, wo_ref[...]) + bo_ref[...]
    x = x_ref[...]
    dr = x_rec - x
    dc = z_q - z_e
    rpart = jnp.sum(dr * dr).reshape(1, 1)
    cpart = jnp.sum(dc * dc).reshape(1, 1)

    @pl.when(pl.program_id(0) == 0)
    def _init():
        rsum_ref[...] = rpart
        csum_ref[...] = cpart

    @pl.when(pl.program_id(0) != 0)
    def _acc():
        rsum_ref[...] += rpart
        csum_ref[...] += cpart


_sc_info = plsc.get_sparse_core_info()
_NC, _NS = _sc_info.num_cores, _sc_info.num_subcores
_NW = _NC * _NS
_B_GATHER = N_TOKENS * K_CAND
_BPW = _B_GATHER // _NW


_TBL_WORDS = N_CODES * D_LAT
_LANES = 16
_GROUPS = _BPW // _LANES


@functools.partial(
    pl.kernel,
    mesh=plsc.VectorSubcoreMesh(core_axis_name="c", subcore_axis_name="s"),
    compiler_params=pltpu.CompilerParams(needs_layout_passes=False),
    out_type=jax.ShapeDtypeStruct((_B_GATHER * D_LAT,), _f32),
    scratch_types=[
        pltpu.VMEM((_TBL_WORDS,), _f32),
        pltpu.VMEM((_BPW,), jnp.int32),
        pltpu.VMEM((_BPW * D_LAT,), _f32),
    ],
)
def _sc_gather(table_hbm, idx_hbm, out_hbm, table_v, idx_v, rows_v):
    wid = lax.axis_index("s") * _NC + lax.axis_index("c")
    pltpu.sync_copy(table_hbm, table_v)
    pltpu.sync_copy(idx_hbm.at[pl.ds(wid * _BPW, _BPW)], idx_v)
    def group(g, carry):
        iv = idx_v[pl.ds(g * _LANES, _LANES)]
        gb = g * (_LANES * D_LAT)
        for l in range(_LANES):
            tb = iv[l] * D_LAT
            ob = gb + l * D_LAT
            for q in range(0, D_LAT, _LANES):
                rows_v[pl.ds(ob + q, _LANES)] = table_v[pl.ds(tb + q, _LANES)]
        return carry

    lax.fori_loop(0, _GROUPS, group, 0)
    pltpu.sync_copy(rows_v, out_hbm.at[pl.ds(wid * (_BPW * D_LAT), _BPW * D_LAT)])


def kernel(x, enc_w1, enc_b1, enc_w2, enc_b2, lat_w, lat_b, codebook,
           dec_w1, dec_b1, dec_w2, dec_b2, out_w, out_b):
    B, S, F = x.shape
    xf = x.reshape(N_TOKENS, F)
    b1 = enc_b1.reshape(1, -1)
    b2 = enc_b2.reshape(1, -1)
    bl = lat_b.reshape(1, -1)
    db1 = dec_b1.reshape(1, -1)
    db2 = dec_b2.reshape(1, -1)
    bo = out_b.reshape(1, -1)

    full = lambda shp: pl.BlockSpec(shp, lambda t: tuple(0 for _ in shp))
    z_e, cand = pl.pallas_call(
        _encode_topk_body,
        grid=(N_BLOCKS,),
        in_specs=[
            pl.BlockSpec((BLK, F), lambda t: (t, 0)),
            full((64, F)), full((1, 64)),
            full((128, 64)), full((1, 128)),
            full((D_LAT, 128)), full((1, D_LAT)),
            full((N_CODES, D_LAT)),
        ],
        out_specs=[
            pl.BlockSpec((BLK, D_LAT), lambda t: (t, 0)),
            pl.BlockSpec((BLK, K_CAND), lambda t: (t, 0)),
        ],
        out_shape=[
            jax.ShapeDtypeStruct((N_TOKENS, D_LAT), _f32),
            jax.ShapeDtypeStruct((N_TOKENS, K_CAND), jnp.int32),
        ],
    )(xf, enc_w1, b1, enc_w2, b2, lat_w, bl, codebook)

    idx_flat = cand.T.reshape(_B_GATHER)
    rows = _sc_gather(codebook.reshape(_TBL_WORDS), idx_flat)
    rows = rows.reshape(K_CAND, N_TOKENS, D_LAT)

    zq, rsum, csum = pl.pallas_call(
        _refine_decode_body,
        grid=(N_BLOCKS,),
        in_specs=[
            pl.BlockSpec((BLK, F), lambda t: (t, 0)),
            pl.BlockSpec((BLK, D_LAT), lambda t: (t, 0)),
            pl.BlockSpec((K_CAND, BLK, D_LAT), lambda t: (0, t, 0)),
            pl.BlockSpec((BLK, K_CAND), lambda t: (t, 0)),
            full((64, D_LAT)), full((1, 64)),
            full((128, 64)), full((1, 128)),
            full((F, 128)), full((1, F)),
        ],
        out_specs=[
            pl.BlockSpec((BLK, D_LAT), lambda t: (t, 0)),
            pl.BlockSpec((1, 1), lambda t: (0, 0)),
            pl.BlockSpec((1, 1), lambda t: (0, 0)),
        ],
        out_shape=[
            jax.ShapeDtypeStruct((N_TOKENS, D_LAT), _f32),
            jax.ShapeDtypeStruct((1, 1), _f32),
            jax.ShapeDtypeStruct((1, 1), _f32),
        ],
    )(xf, z_e, rows, cand, dec_w1, db1, dec_w2, db2, out_w, bo)

    z_latent = zq.reshape(B, S, D_LAT)
    recon_loss = rsum[0, 0] / jnp.float32(N_TOKENS * F)
    commit = csum[0, 0] / jnp.float32(N_TOKENS * D_LAT)
    loss = recon_loss + 0.25 * commit + 0.25 * commit
    return (z_latent, loss)


# speedup vs baseline: 1.1784x; 1.0543x over previous
import functools

import jax
import jax.numpy as jnp
import numpy as np
from jax import lax
from jax.experimental import pallas as pl
from jax.experimental.pallas import tpu as pltpu
from jax.experimental.pallas import tpu_sc as plsc

N_TOKENS = 4608
BLK = 1152
N_BLOCKS = N_TOKENS // BLK
K_CAND = 2
N_CODES = 1024
D_LAT = 64

_f32 = jnp.float32
_bf16 = jnp.bfloat16


def _mm1(a, b):
    return lax.dot_general(a.astype(_bf16), b.astype(_bf16),
                           (((1,), (1,)), ((), ())),
                           preferred_element_type=_f32)


def _c(v):
    return jnp.float32(v)


def _erfc(w):
    one = _c(1.0)
    ax = jnp.abs(w)
    z = w * w
    p = z * _c(7.85386146e-05)
    for c in (-0.000801019371, 0.00518832775, -0.0268538129, 0.112835854,
              -0.37612626):
        p = (p + _c(c)) * z
    p = p + _c(1.12837911)
    small = one - w * p
    nz = -z
    e = jnp.exp(nz)
    q = e * (one / ax)
    r = one / z
    pa = r * _c(0.0232682)
    for c in (-0.138703942, 0.368742466, -0.582473278, 0.621000469,
              -0.494451523, 0.340488, -0.274112701):
        pa = (pa + _c(c)) * r
    pa = pa + _c(0.563825965)
    pb = r * _c(-10.477664)
    for c in (12.9772, -7.49551868, 2.92101908, -1.01526523, 0.42184633,
              -0.282076746):
        pb = (pb + _c(c)) * r
    pb = pb + _c(0.564189494)
    big = q * jnp.where(ax < _c(2.0), pa, pb)
    big = jnp.where(nz < _c(-88.7228394), _c(0.0), big)
    big = jnp.where(w < _c(0.0), _c(2.0) - big, big)
    return jnp.where(ax < one, small, big)


def _gelu(x):
    return (x * _c(0.5)) * _erfc((-x) * _c(0.70710676908493042))


def _encode_topk_body(x_ref, w1_ref, b1_ref, w2_ref, b2_ref, wl_ref, bl_ref,
                      cb_ref, ze_ref, cand_ref):
    x = x_ref[...]
    h = _gelu(_mm1(x, w1_ref[...]) + b1_ref[...])
    h = _gelu(_mm1(h, w2_ref[...]) + b2_ref[...])
    z_e = _mm1(h, wl_ref[...]) + bl_ref[...]
    ze_ref[...] = z_e

    cb = cb_ref[...]
    cb2 = cb * cb
    c2h = cb2.astype(_bf16)
    c2l = (cb2 - c2h.astype(_f32)).astype(_bf16)
    ones = jnp.ones((1, 2 * D_LAT), _bf16)
    cn2 = lax.dot_general(ones, jnp.concatenate([c2h, c2l], axis=1),
                          (((1,), (1,)), ((), ())),
                          preferred_element_type=_f32)
    zh = z_e.astype(_bf16)
    zl = (z_e - zh.astype(_f32)).astype(_bf16)
    ch = cb.astype(_bf16)
    cl = (cb - ch.astype(_f32)).astype(_bf16)
    za = jnp.concatenate([zh, zl, zh], axis=1)
    ca = jnp.concatenate([ch, ch, cl], axis=1)
    s = lax.dot_general(za, ca, (((1,), (1,)), ((), ())),
                        preferred_element_type=_f32)
    dist = cn2 - 2.0 * s
    iota = lax.broadcasted_iota(jnp.int32, (BLK, N_CODES), 1)
    big = jnp.int32(1 << 30)
    cands = []
    work = dist
    for j in range(K_CAND):
        m = jnp.min(work, axis=1, keepdims=True)
        idxj = jnp.min(jnp.where(work == m, iota, big), axis=1, keepdims=True)
        cands.append(idxj)
        if j + 1 < K_CAND:
            work = jnp.where(iota == idxj, jnp.float32(np.inf), work)
    cand_ref[...] = jnp.concatenate(cands, axis=1)


def _refine_decode_body(x_ref, ze_ref, rows_ref, cand_ref,
                        d1_ref, db1_ref, d2_ref, db2_ref, wo_ref, bo_ref,
                        zq_ref, rsum_ref, csum_ref):
    z_e = ze_ref[...]
    best_d = jnp.full((BLK, 1), np.inf, _f32)
    best_i = jnp.full((BLK, 1), 1 << 30, jnp.int32)
    z_q = jnp.zeros((BLK, D_LAT), _f32)
    for j in range(K_CAND):
        row = rows_ref[j]
        idx = cand_ref[:, j:j + 1]
        diff = z_e - row
        dj = jnp.sqrt(jnp.sum(diff * diff, axis=1, keepdims=True))
        better = (dj < best_d) | ((dj == best_d) & (idx < best_i))
        best_d = jnp.where(better, dj, best_d)
        best_i = jnp.where(better, idx, best_i)
        z_q = jnp.where(better, row, z_q)
    zq_ref[...] = z_q

    r = jax.nn.gelu(_mm1(z_q, d1_ref[...]) + db1_ref[...], approximate=True)
    r = jax.nn.gelu(_mm1(r, d2_ref[...]) + db2_ref[...], approximate=True)
    x_rec = _mm1(r, wo_ref[...]) + bo_ref[...]
    x = x_ref[...]
    dr = x_rec - x
    dc = z_q - z_e
    rpart = jnp.sum(dr * dr).reshape(1, 1)
    cpart = jnp.sum(dc * dc).reshape(1, 1)

    @pl.when(pl.program_id(0) == 0)
    def _init():
        rsum_ref[...] = rpart
        csum_ref[...] = cpart

    @pl.when(pl.program_id(0) != 0)
    def _acc():
        rsum_ref[...] += rpart
        csum_ref[...] += cpart


_sc_info = plsc.get_sparse_core_info()
_NC, _NS = _sc_info.num_cores, _sc_info.num_subcores
_NW = _NC * _NS
_B_GATHER = N_TOKENS * K_CAND
_BPW = _B_GATHER // _NW


_TBL_WORDS = N_CODES * D_LAT
_LANES = 16
_GROUPS = _BPW // _LANES


@functools.partial(
    pl.kernel,
    mesh=plsc.VectorSubcoreMesh(core_axis_name="c", subcore_axis_name="s"),
    compiler_params=pltpu.CompilerParams(needs_layout_passes=False),
    out_type=jax.ShapeDtypeStruct((_B_GATHER * D_LAT,), _f32),
    scratch_types=[
        pltpu.VMEM((_TBL_WORDS,), _f32),
        pltpu.VMEM((_BPW,), jnp.int32),
        pltpu.VMEM((_BPW * D_LAT,), _f32),
    ],
)
def _sc_gather(table_hbm, idx_hbm, out_hbm, table_v, idx_v, rows_v):
    wid = lax.axis_index("s") * _NC + lax.axis_index("c")
    pltpu.sync_copy(table_hbm, table_v)
    pltpu.sync_copy(idx_hbm.at[pl.ds(wid * _BPW, _BPW)], idx_v)
    def group(g, carry):
        iv = idx_v[pl.ds(g * _LANES, _LANES)]
        gb = g * (_LANES * D_LAT)
        for l in range(_LANES):
            tb = iv[l] * D_LAT
            ob = gb + l * D_LAT
            for q in range(0, D_LAT, _LANES):
                rows_v[pl.ds(ob + q, _LANES)] = table_v[pl.ds(tb + q, _LANES)]
        return carry

    lax.fori_loop(0, _GROUPS, group, 0)
    pltpu.sync_copy(rows_v, out_hbm.at[pl.ds(wid * (_BPW * D_LAT), _BPW * D_LAT)])


def kernel(x, enc_w1, enc_b1, enc_w2, enc_b2, lat_w, lat_b, codebook,
           dec_w1, dec_b1, dec_w2, dec_b2, out_w, out_b):
    B, S, F = x.shape
    xf = x.reshape(N_TOKENS, F)
    b1 = enc_b1.reshape(1, -1)
    b2 = enc_b2.reshape(1, -1)
    bl = lat_b.reshape(1, -1)
    db1 = dec_b1.reshape(1, -1)
    db2 = dec_b2.reshape(1, -1)
    bo = out_b.reshape(1, -1)

    full = lambda shp: pl.BlockSpec(shp, lambda t: tuple(0 for _ in shp))
    z_e, cand = pl.pallas_call(
        _encode_topk_body,
        grid=(N_BLOCKS,),
        in_specs=[
            pl.BlockSpec((BLK, F), lambda t: (t, 0)),
            full((64, F)), full((1, 64)),
            full((128, 64)), full((1, 128)),
            full((D_LAT, 128)), full((1, D_LAT)),
            full((N_CODES, D_LAT)),
        ],
        out_specs=[
            pl.BlockSpec((BLK, D_LAT), lambda t: (t, 0)),
            pl.BlockSpec((BLK, K_CAND), lambda t: (t, 0)),
        ],
        out_shape=[
            jax.ShapeDtypeStruct((N_TOKENS, D_LAT), _f32),
            jax.ShapeDtypeStruct((N_TOKENS, K_CAND), jnp.int32),
        ],
    )(xf, enc_w1, b1, enc_w2, b2, lat_w, bl, codebook)

    idx_flat = cand.T.reshape(_B_GATHER)
    rows = _sc_gather(codebook.reshape(_TBL_WORDS), idx_flat)
    rows = rows.reshape(K_CAND, N_TOKENS, D_LAT)

    zq, rsum, csum = pl.pallas_call(
        _refine_decode_body,
        grid=(N_BLOCKS,),
        in_specs=[
            pl.BlockSpec((BLK, F), lambda t: (t, 0)),
            pl.BlockSpec((BLK, D_LAT), lambda t: (t, 0)),
            pl.BlockSpec((K_CAND, BLK, D_LAT), lambda t: (0, t, 0)),
            pl.BlockSpec((BLK, K_CAND), lambda t: (t, 0)),
            full((64, D_LAT)), full((1, 64)),
            full((128, 64)), full((1, 128)),
            full((F, 128)), full((1, F)),
        ],
        out_specs=[
            pl.BlockSpec((BLK, D_LAT), lambda t: (t, 0)),
            pl.BlockSpec((1, 1), lambda t: (0, 0)),
            pl.BlockSpec((1, 1), lambda t: (0, 0)),
        ],
        out_shape=[
            jax.ShapeDtypeStruct((N_TOKENS, D_LAT), _f32),
            jax.ShapeDtypeStruct((1, 1), _f32),
            jax.ShapeDtypeStruct((1, 1), _f32),
        ],
    )(xf, z_e, rows, cand, dec_w1, db1, dec_w2, db2, out_w, bo)

    z_latent = zq.reshape(B, S, D_LAT)
    recon_loss = rsum[0, 0] / jnp.float32(N_TOKENS * F)
    commit = csum[0, 0] / jnp.float32(N_TOKENS * D_LAT)
    loss = recon_loss + 0.25 * commit + 0.25 * commit
    return (z_latent, loss)
